# baseline (speedup 1.0000x reference)
import jax
import jax.numpy as jnp
from jax import lax
from jax.experimental import pallas as pl
from jax.experimental.pallas import tpu as pltpu

B, P, F = 16, 1024, 16
PROJ = 32
HEADS = 2
KD = PROJ // HEADS
K = 16
EPS = 0.001
_BIG = 1e9
_INV_SQRT_KD = 1.0 / (KD ** 0.5)
EV = 2


def _gelu(v):
    return 0.5 * v * (1.0 + lax.erf(v * 0.7071067811865476))


def _gn(e, gamma, beta):
    mu = jnp.mean(e)
    var = jnp.mean((e - mu) ** 2)
    return (e - mu) * lax.rsqrt(var + EPS) * gamma + beta


def _knn_multi(ptss, feats, w1, b1, w2, b2):
    c = feats[0].shape[1]
    NT = 8
    TW = P // NT
    cols = lax.broadcasted_iota(jnp.int32, (P, P), 1)
    rows = lax.broadcasted_iota(jnp.int32, (P, P), 0)
    lane = lax.broadcasted_iota(jnp.int32, (P, TW), 1)
    tile_iota = lax.broadcasted_iota(jnp.int32, (P, NT), 1)
    w1t = w1[:c]
    ds, bases, tmins = [], [], []
    for pts, feat in zip(ptss, feats):
        r = jnp.sum(pts * pts, axis=1, keepdims=True)
        m = jnp.dot(pts, pts.T, preferred_element_type=jnp.float32)
        d = jnp.abs(r - 2.0 * m + r.T)
        d = jnp.where(rows == cols, _BIG, d)
        ds.append(d)
        bases.append(jnp.dot(feat, w1[c:], preferred_element_type=jnp.float32) + b1)
        tmins.append(jnp.concatenate(
            [jnp.min(d[:, t * TW:(t + 1) * TW], axis=1, keepdims=True)
             for t in range(NT)], axis=1))

    def body(_, carry):
        tms, rbs, accs = carry
        ntms, nrbs, naccs = [], [], []
        for d, tm, rb, acc, feat, base in zip(ds, tms, rbs, accs, feats, bases):
            mn8 = jnp.min(tm, axis=1, keepdims=True)
            tstar = jnp.min(jnp.where(tm <= mn8, tile_iota, NT),
                            axis=1, keepdims=True)
            dsel = jnp.zeros((P, TW), jnp.float32)
            for t in range(NT):
                dsel = dsel + jnp.where(tstar == t,
                                        d[:, t * TW:(t + 1) * TW], 0.0)
            removed = jnp.bitwise_and(jnp.right_shift(rb, tstar), 1)
            dsel = jnp.where(removed == 1, _BIG, dsel)
            mnl = jnp.min(dsel, axis=1, keepdims=True)
            lstar = jnp.min(jnp.where(dsel <= mnl, lane, TW),
                            axis=1, keepdims=True)
            hot = lane == lstar
            nextmin = jnp.min(jnp.where(hot, _BIG, dsel), axis=1, keepdims=True)
            ntms.append(jnp.where(tile_iota == tstar, nextmin, tm))
            nrbs.append(jnp.bitwise_or(
                rb, jnp.left_shift(hot.astype(jnp.int32), tstar)))
            oh = jnp.where(hot, 1.0, 0.0)
            g = jnp.zeros((P, c), jnp.float32)
            for t in range(NT):
                a_t = jnp.dot(oh, feat[t * TW:(t + 1) * TW],
                              preferred_element_type=jnp.float32)
                g = g + jnp.where(tstar == t, a_t, 0.0)
            h = _gelu(jnp.dot(g - feat, w1t, preferred_element_type=jnp.float32)
                      + base)
            h = _gelu(jnp.dot(h, w2, preferred_element_type=jnp.float32) + b2)
            naccs.append(acc + h)
        return tuple(ntms), tuple(nrbs), tuple(naccs)

    zeros = tuple(jnp.zeros((P, PROJ), jnp.float32) for _ in feats)
    rb0 = tuple(jnp.zeros((P, TW), jnp.int32) for _ in feats)
    _, _, accs = lax.fori_loop(0, K, body, (tuple(tmins), rb0, zeros))
    return [a * (1.0 / K) for a in accs]


def _attn_full(x1, wq, bq, wk, bk, wv, bv, wo, bo):
    q = (jnp.dot(x1, wq, preferred_element_type=jnp.float32) + bq) * _INV_SQRT_KD
    k = jnp.dot(x1, wk, preferred_element_type=jnp.float32) + bk
    v = jnp.dot(x1, wv, preferred_element_type=jnp.float32) + bv
    o = bo
    for h in range(HEADS):
        sl = slice(h * KD, (h + 1) * KD)
        lg = jnp.dot(q[:, sl], k[:, sl].T, preferred_element_type=jnp.float32)
        mx = jnp.max(lg, axis=1, keepdims=True)
        ex = jnp.exp(lg - mx)
        sm = jnp.sum(ex, axis=1, keepdims=True)
        oh = jnp.dot(ex, v[:, sl], preferred_element_type=jnp.float32) / sm
        o = o + jnp.dot(oh, wo[sl], preferred_element_type=jnp.float32)
    return o


def _block(encoded, mask, bw):
    (g1, bt1, wq, bq, wk, bk, wv, bv, wo, bo,
     ls1, g2, bt2, fw1, fb1, fw2, fb2, ls2) = bw
    x1 = _gn(encoded, g1, bt1)
    upd = _attn_full(x1, wq, bq, wk, bk, wv, bv, wo, bo) * ls1 * mask
    x2 = upd + encoded
    x3 = _gn(x2, g2, bt2)
    x3 = _gelu(jnp.dot(x3, fw1, preferred_element_type=jnp.float32) + fb1)
    x3 = jnp.dot(x3, fw2, preferred_element_type=jnp.float32) + fb2
    x3 = x3 * ls2 * mask
    return (x3 + x2) * mask


def _head_layer(ct, body, g1, bt1, wq, bq, wk, bk, wv, bv, wo, bo, g2, bt2, ls):
    n = (P + 1) * PROJ
    mu = (jnp.sum(body) + jnp.sum(ct)) / n
    var = (jnp.sum((body - mu) ** 2) + jnp.sum((ct - mu) ** 2)) / n
    inv = lax.rsqrt(var + EPS)
    bn = (body - mu) * inv * g1 + bt1
    cn = (ct - mu) * inv * g1 + bt1
    q = (jnp.dot(cn, wq, preferred_element_type=jnp.float32) + bq) * _INV_SQRT_KD
    kb = jnp.dot(bn, wk, preferred_element_type=jnp.float32) + bk
    kc = jnp.dot(cn, wk, preferred_element_type=jnp.float32) + bk
    vb = jnp.dot(bn, wv, preferred_element_type=jnp.float32) + bv
    vc = jnp.dot(cn, wv, preferred_element_type=jnp.float32) + bv
    u = bo
    for h in range(HEADS):
        sl = slice(h * KD, (h + 1) * KD)
        lg = jnp.dot(q[:, sl], kb[:, sl].T,
                     preferred_element_type=jnp.float32)
        lc = jnp.sum(q[:, sl] * kc[:, sl])
        mx = jnp.maximum(jnp.max(lg), lc)
        ex = jnp.exp(lg - mx)
        ec = jnp.exp(lc - mx)
        den = jnp.sum(ex) + ec
        oh = (jnp.dot(ex, vb[:, sl], preferred_element_type=jnp.float32)
              + ec * vc[:, sl]) / den
        u = u + jnp.dot(oh, wo[sl], preferred_element_type=jnp.float32)
    m2 = jnp.mean(u)
    v2 = jnp.mean((u - m2) ** 2)
    u = (u - m2) * lax.rsqrt(v2 + EPS) * g2 + bt2
    return u * ls + ct


def _pet_kernel(x_ref, *refs):
    out_ref = refs[-1]
    w = [r[...] for r in refs[:-1]]

    xs = [x_ref[e] for e in range(EV)]
    masks = [(xe[:, 0:1] != 0.0).astype(jnp.float32) for xe in xs]
    shifts = [999.0 * (1.0 - m) for m in masks]

    encs = []
    for xe in xs:
        enc = _gelu(jnp.dot(xe, w[0], preferred_element_type=jnp.float32) + w[1])
        encs.append(_gelu(jnp.dot(enc, w[2], preferred_element_type=jnp.float32)
                          + w[3]))

    feats = xs
    ptss = [xe[:, :2] + s for xe, s in zip(xs, shifts)]
    for layer in range(2):
        lw = w[4 + 4 * layer: 8 + 4 * layer]
        feats = _knn_multi(ptss, feats, *lw)
        ptss = [f + s for f, s in zip(feats, shifts)]

    encoded = [f + e for f, e in zip(feats, encs)]
    skips = [e * m for e, m in zip(encoded, masks)]

    for blk in range(2):
        bw = w[12 + 18 * blk: 30 + 18 * blk]
        encoded = [_block(e, m, bw) for e, m in zip(encoded, masks)]

    bodies = [e + s for e, s in zip(encoded, skips)]

    cts = [w[48] for _ in range(EV)]
    for hl in range(2):
        hw = w[49 + 13 * hl: 62 + 13 * hl]
        cts = [_head_layer(ct, b, *hw) for ct, b in zip(cts, bodies)]

    fg, fb = w[75], w[76]
    ow, ob = w[77], w[78]
    for e, ct in enumerate(cts):
        m = jnp.mean(ct)
        v = jnp.mean((ct - m) ** 2)
        ctn = (ct - m) * lax.rsqrt(v + EPS) * fg + fb
        out_ref[e] = jnp.dot(ctn, ow, preferred_element_type=jnp.float32) + ob


def _rb(a):
    return a.reshape(1, -1)


def _mha_flat(mp):
    return [mp['wq'].reshape(PROJ, PROJ), _rb(mp['bq']),
            mp['wk'].reshape(PROJ, PROJ), _rb(mp['bk']),
            mp['wv'].reshape(PROJ, PROJ), _rb(mp['bv']),
            mp['wo'].reshape(PROJ, PROJ), _rb(mp['bo'])]


def _flatten_params(p):
    flat = [p['enc']['w1'], _rb(p['enc']['b1']), p['enc']['w2'], _rb(p['enc']['b2'])]
    for lp in p['local']:
        flat += [lp['w1'], _rb(lp['b1']), lp['w2'], _rb(lp['b2'])]
    for bp in p['blocks']:
        flat += [_rb(bp['gn1']['gamma']), _rb(bp['gn1']['beta'])]
        flat += _mha_flat(bp['mha'])
        flat += [_rb(bp['ls1']), _rb(bp['gn2']['gamma']), _rb(bp['gn2']['beta']),
                 bp['ff']['w1'], _rb(bp['ff']['b1']),
                 bp['ff']['w2'], _rb(bp['ff']['b2']), _rb(bp['ls2'])]
    flat.append(p['class_token'])
    for hp in p['head']:
        flat += [_rb(hp['gn1']['gamma']), _rb(hp['gn1']['beta'])]
        flat += _mha_flat(hp['mha'])
        flat += [_rb(hp['gn2']['gamma']), _rb(hp['gn2']['beta']), _rb(hp['ls'])]
    flat += [_rb(p['final_gn']['gamma']), _rb(p['final_gn']['beta']),
             p['out']['w'], _rb(p['out']['b'])]
    return flat


def kernel(x, params):
    flat = _flatten_params(params)
    w_specs = [pl.BlockSpec(a.shape, lambda b, n=a.ndim: (0,) * n) for a in flat]
    out = pl.pallas_call(
        _pet_kernel,
        grid=(B // EV,),
        in_specs=[pl.BlockSpec((EV, P, F), lambda b: (b, 0, 0))] + w_specs,
        out_specs=pl.BlockSpec((EV, 1, 2), lambda b: (b, 0, 0)),
        out_shape=jax.ShapeDtypeStruct((B, 1, 2), jnp.float32),
        compiler_params=pltpu.CompilerParams(
            dimension_semantics=("parallel",)),
    )(x, *flat)
    return out[:, 0, :]


# speedup vs baseline: 1.0325x; 1.0325x over previous
import jax
import jax.numpy as jnp
from jax import lax
from jax.experimental import pallas as pl
from jax.experimental.pallas import tpu as pltpu

B, P, F = 16, 1024, 16
PROJ = 32
HEADS = 2
KD = PROJ // HEADS
K = 16
EPS = 0.001
_BIG = 1e9
_INV_SQRT_KD = 1.0 / (KD ** 0.5)
EV = 1


def _gelu(v):
    return 0.5 * v * (1.0 + lax.erf(v * 0.7071067811865476))


def _gn(e, gamma, beta):
    mu = jnp.mean(e)
    var = jnp.mean((e - mu) ** 2)
    return (e - mu) * lax.rsqrt(var + EPS) * gamma + beta


def _knn_multi(ptss, feats, w1, b1, w2, b2):
    c = feats[0].shape[1]
    NT = 8
    TW = P // NT
    cols = lax.broadcasted_iota(jnp.int32, (P, P), 1)
    rows = lax.broadcasted_iota(jnp.int32, (P, P), 0)
    lane = lax.broadcasted_iota(jnp.int32, (P, TW), 1)
    tile_iota = lax.broadcasted_iota(jnp.int32, (P, NT), 1)
    w1t = w1[:c]
    ds, bases, tmins = [], [], []
    for pts, feat in zip(ptss, feats):
        r = jnp.sum(pts * pts, axis=1, keepdims=True)
        m = jnp.dot(pts, pts.T, preferred_element_type=jnp.float32)
        d = jnp.abs(r - 2.0 * m + r.T)
        d = jnp.where(rows == cols, _BIG, d)
        ds.append(d)
        bases.append(jnp.dot(feat, w1[c:], preferred_element_type=jnp.float32) + b1)
        tmins.append(jnp.concatenate(
            [jnp.min(d[:, t * TW:(t + 1) * TW], axis=1, keepdims=True)
             for t in range(NT)], axis=1))

    def body(_, carry):
        tms, rbs, accs = carry
        ntms, nrbs, naccs = [], [], []
        for d, tm, rb, acc, feat, base in zip(ds, tms, rbs, accs, feats, bases):
            mn8 = jnp.min(tm, axis=1, keepdims=True)
            tstar = jnp.min(jnp.where(tm <= mn8, tile_iota, NT),
                            axis=1, keepdims=True)
            dsel = jnp.zeros((P, TW), jnp.float32)
            for t in range(NT):
                dsel = dsel + jnp.where(tstar == t,
                                        d[:, t * TW:(t + 1) * TW], 0.0)
            removed = jnp.bitwise_and(jnp.right_shift(rb, tstar), 1)
            dsel = jnp.where(removed == 1, _BIG, dsel)
            mnl = jnp.min(dsel, axis=1, keepdims=True)
            lstar = jnp.min(jnp.where(dsel <= mnl, lane, TW),
                            axis=1, keepdims=True)
            hot = lane == lstar
            nextmin = jnp.min(jnp.where(hot, _BIG, dsel), axis=1, keepdims=True)
            ntms.append(jnp.where(tile_iota == tstar, nextmin, tm))
            nrbs.append(jnp.bitwise_or(
                rb, jnp.left_shift(hot.astype(jnp.int32), tstar)))
            oh = jnp.where(hot, 1.0, 0.0)
            g = jnp.zeros((P, c), jnp.float32)
            for t in range(NT):
                a_t = jnp.dot(oh, feat[t * TW:(t + 1) * TW],
                              preferred_element_type=jnp.float32)
                g = g + jnp.where(tstar == t, a_t, 0.0)
            h = _gelu(jnp.dot(g - feat, w1t, preferred_element_type=jnp.float32)
                      + base)
            h = _gelu(jnp.dot(h, w2, preferred_element_type=jnp.float32) + b2)
            naccs.append(acc + h)
        return tuple(ntms), tuple(nrbs), tuple(naccs)

    zeros = tuple(jnp.zeros((P, PROJ), jnp.float32) for _ in feats)
    rb0 = tuple(jnp.zeros((P, TW), jnp.int32) for _ in feats)
    _, _, accs = lax.fori_loop(0, K, body, (tuple(tmins), rb0, zeros))
    return [a * (1.0 / K) for a in accs]


def _attn_full(x1, wqkv, bqkv, wo, bo):
    qkv = jnp.dot(x1, wqkv, preferred_element_type=jnp.float32) + bqkv
    q = qkv[:, :PROJ] * _INV_SQRT_KD
    k = qkv[:, PROJ:2 * PROJ]
    v = qkv[:, 2 * PROJ:]
    o = bo
    for h in range(HEADS):
        sl = slice(h * KD, (h + 1) * KD)
        lg = jnp.dot(q[:, sl], k[:, sl].T, preferred_element_type=jnp.float32)
        mx = jnp.max(lg, axis=1, keepdims=True)
        ex = jnp.exp(lg - mx)
        sm = jnp.sum(ex, axis=1, keepdims=True)
        oh = jnp.dot(ex, v[:, sl], preferred_element_type=jnp.float32) / sm
        o = o + jnp.dot(oh, wo[sl], preferred_element_type=jnp.float32)
    return o


def _block(encoded, mask, bw):
    (g1, bt1, wqkv, bqkv, wo, bo,
     ls1, g2, bt2, fw1, fb1, fw2, fb2, ls2) = bw
    x1 = _gn(encoded, g1, bt1)
    upd = _attn_full(x1, wqkv, bqkv, wo, bo) * ls1 * mask
    x2 = upd + encoded
    x3 = _gn(x2, g2, bt2)
    x3 = _gelu(jnp.dot(x3, fw1, preferred_element_type=jnp.float32) + fb1)
    x3 = jnp.dot(x3, fw2, preferred_element_type=jnp.float32) + fb2
    x3 = x3 * ls2 * mask
    return (x3 + x2) * mask


def _head_layer(ct, body, g1, bt1, wqkv, bqkv, wo, bo, g2, bt2, ls):
    n = (P + 1) * PROJ
    mu = (jnp.sum(body) + jnp.sum(ct)) / n
    var = (jnp.sum((body - mu) ** 2) + jnp.sum((ct - mu) ** 2)) / n
    inv = lax.rsqrt(var + EPS)
    bn = (body - mu) * inv * g1 + bt1
    cn = (ct - mu) * inv * g1 + bt1
    qkv_c = jnp.dot(cn, wqkv, preferred_element_type=jnp.float32) + bqkv
    q = qkv_c[:, :PROJ] * _INV_SQRT_KD
    kc = qkv_c[:, PROJ:2 * PROJ]
    vc = qkv_c[:, 2 * PROJ:]
    kvb = (jnp.dot(bn, wqkv[:, PROJ:], preferred_element_type=jnp.float32)
           + bqkv[:, PROJ:])
    kb = kvb[:, :PROJ]
    vb = kvb[:, PROJ:]
    u = bo
    for h in range(HEADS):
        sl = slice(h * KD, (h + 1) * KD)
        lg = jnp.dot(q[:, sl], kb[:, sl].T,
                     preferred_element_type=jnp.float32)
        lc = jnp.sum(q[:, sl] * kc[:, sl])
        mx = jnp.maximum(jnp.max(lg), lc)
        ex = jnp.exp(lg - mx)
        ec = jnp.exp(lc - mx)
        den = jnp.sum(ex) + ec
        oh = (jnp.dot(ex, vb[:, sl], preferred_element_type=jnp.float32)
              + ec * vc[:, sl]) / den
        u = u + jnp.dot(oh, wo[sl], preferred_element_type=jnp.float32)
    m2 = jnp.mean(u)
    v2 = jnp.mean((u - m2) ** 2)
    u = (u - m2) * lax.rsqrt(v2 + EPS) * g2 + bt2
    return u * ls + ct


def _pet_kernel(x_ref, *refs):
    out_ref = refs[-1]
    w = [r[...] for r in refs[:-1]]

    xs = [x_ref[e] for e in range(EV)]
    masks = [(xe[:, 0:1] != 0.0).astype(jnp.float32) for xe in xs]
    shifts = [999.0 * (1.0 - m) for m in masks]

    encs = []
    for xe in xs:
        enc = _gelu(jnp.dot(xe, w[0], preferred_element_type=jnp.float32) + w[1])
        encs.append(_gelu(jnp.dot(enc, w[2], preferred_element_type=jnp.float32)
                          + w[3]))

    feats = xs
    ptss = [xe[:, :2] + s for xe, s in zip(xs, shifts)]
    for layer in range(2):
        lw = w[4 + 4 * layer: 8 + 4 * layer]
        feats = _knn_multi(ptss, feats, *lw)
        ptss = [f + s for f, s in zip(feats, shifts)]

    encoded = [f + e for f, e in zip(feats, encs)]
    skips = [e * m for e, m in zip(encoded, masks)]

    for blk in range(2):
        bw = w[12 + 14 * blk: 26 + 14 * blk]
        encoded = [_block(e, m, bw) for e, m in zip(encoded, masks)]

    bodies = [e + s for e, s in zip(encoded, skips)]

    cts = [w[40] for _ in range(EV)]
    for hl in range(2):
        hw = w[41 + 9 * hl: 50 + 9 * hl]
        cts = [_head_layer(ct, b, *hw) for ct, b in zip(cts, bodies)]

    fg, fb = w[59], w[60]
    ow, ob = w[61], w[62]
    for e, ct in enumerate(cts):
        m = jnp.mean(ct)
        v = jnp.mean((ct - m) ** 2)
        ctn = (ct - m) * lax.rsqrt(v + EPS) * fg + fb
        out_ref[e] = jnp.dot(ctn, ow, preferred_element_type=jnp.float32) + ob


def _rb(a):
    return a.reshape(1, -1)


def _mha_flat(mp):
    wqkv = jnp.concatenate([mp['wq'].reshape(PROJ, PROJ),
                            mp['wk'].reshape(PROJ, PROJ),
                            mp['wv'].reshape(PROJ, PROJ)], axis=1)
    bqkv = jnp.concatenate([_rb(mp['bq']), _rb(mp['bk']), _rb(mp['bv'])], axis=1)
    return [wqkv, bqkv, mp['wo'].reshape(PROJ, PROJ), _rb(mp['bo'])]


def _flatten_params(p):
    flat = [p['enc']['w1'], _rb(p['enc']['b1']), p['enc']['w2'], _rb(p['enc']['b2'])]
    for lp in p['local']:
        flat += [lp['w1'], _rb(lp['b1']), lp['w2'], _rb(lp['b2'])]
    for bp in p['blocks']:
        flat += [_rb(bp['gn1']['gamma']), _rb(bp['gn1']['beta'])]
        flat += _mha_flat(bp['mha'])
        flat += [_rb(bp['ls1']), _rb(bp['gn2']['gamma']), _rb(bp['gn2']['beta']),
                 bp['ff']['w1'], _rb(bp['ff']['b1']),
                 bp['ff']['w2'], _rb(bp['ff']['b2']), _rb(bp['ls2'])]
    flat.append(p['class_token'])
    for hp in p['head']:
        flat += [_rb(hp['gn1']['gamma']), _rb(hp['gn1']['beta'])]
        flat += _mha_flat(hp['mha'])
        flat += [_rb(hp['gn2']['gamma']), _rb(hp['gn2']['beta']), _rb(hp['ls'])]
    flat += [_rb(p['final_gn']['gamma']), _rb(p['final_gn']['beta']),
             p['out']['w'], _rb(p['out']['b'])]
    return flat


def kernel(x, params):
    flat = _flatten_params(params)
    w_specs = [pl.BlockSpec(a.shape, lambda b, n=a.ndim: (0,) * n) for a in flat]
    out = pl.pallas_call(
        _pet_kernel,
        grid=(B // EV,),
        in_specs=[pl.BlockSpec((EV, P, F), lambda b: (b, 0, 0))] + w_specs,
        out_specs=pl.BlockSpec((EV, 1, 2), lambda b: (b, 0, 0)),
        out_shape=jax.ShapeDtypeStruct((B, 1, 2), jnp.float32),
        compiler_params=pltpu.CompilerParams(
            dimension_semantics=("parallel",)),
    )(x, *flat)
    return out[:, 0, :]
